# Initial kernel scaffold; baseline (speedup 1.0000x reference)
import jax
import jax.numpy as jnp
from jax.experimental import pallas as pl


def kernel(x, attr, mask):
    raise NotImplementedError("write your pallas kernel here")


# speedup vs baseline: 4240.8962x; 4240.8962x over previous
import functools

import jax
import jax.numpy as jnp
import numpy as np
from jax.experimental import pallas as pl
from jax.experimental.pallas import tpu as pltpu


def _infidelity_kernel(x_ref, attr_ref, out_ref, s_acc, death_acc, *, NP, PS, C, L):
    p = pl.program_id(1)

    a = attr_ref[0]
    pooled = jnp.concatenate(
        [jnp.max(a[:, q * PS:(q + 1) * PS], axis=1, keepdims=True) for q in range(NP)],
        axis=1,
    )

    r = jnp.zeros((PS, NP), dtype=jnp.int32)
    lane = jax.lax.broadcasted_iota(jnp.int32, (1, NP), 1)
    for q in range(NP):
        vq = pooled[:, q:q + 1]
        later = lane > q
        cond = (vq < pooled) | ((vq == pooled) & later)
        r = r + cond.astype(jnp.int32)

    block_death = jnp.min(r, axis=0, keepdims=True)

    @pl.when(p == 0)
    def _():
        death_acc[...] = block_death

    @pl.when(p > 0)
    def _():
        death_acc[...] = jnp.minimum(death_acc[...], block_death)

    s_acc[pl.ds(p, 1), :] = jnp.sum(x_ref[0], axis=0, keepdims=True)

    @pl.when(p == NP - 1)
    def _():
        death = death_acc[...]
        stage = jax.lax.broadcasted_iota(jnp.int32, (NP, NP), 0)
        alive = (death >= stage).astype(jnp.float32)
        m = jnp.dot(alive, s_acc[...], preferred_element_type=jnp.float32)
        m = m * (1.0 / C)

        m = m - jnp.max(m, axis=1, keepdims=True)
        e = jnp.exp(m)
        o = e / jnp.sum(e, axis=1, keepdims=True)

        u = 1.0 / L
        numer = jnp.sum(o[1:, :], axis=0, keepdims=True) + 0.5 * u
        res = (0.5 + numer / o[0:1, :]) * (1.0 / (NP + 1))
        out_ref[0] = res


@jax.jit
def kernel(x, attr, mask):
    B, C, L = x.shape
    PS = int(0.1 * L)
    NP = L // PS

    grid = (B, NP)
    out = pl.pallas_call(
        functools.partial(_infidelity_kernel, NP=NP, PS=PS, C=C, L=L),
        grid=grid,
        in_specs=[
            pl.BlockSpec((1, PS, L), lambda b, p: (b, p, 0)),
            pl.BlockSpec((1, PS, L), lambda b, p: (b, p, 0)),
        ],
        out_specs=pl.BlockSpec((1, 1, L), lambda b, p: (b, 0, 0)),
        out_shape=jax.ShapeDtypeStruct((B, 1, L), jnp.float32),
        scratch_shapes=[
            pltpu.VMEM((NP, L), jnp.float32),
            pltpu.VMEM((1, NP), jnp.int32),
        ],
        compiler_params=pltpu.CompilerParams(
            dimension_semantics=("parallel", "arbitrary"),
        ),
    )(x, attr)
    return out.reshape(B, L)


# speedup vs baseline: 4808.0596x; 1.1337x over previous
import functools

import jax
import jax.numpy as jnp
import numpy as np
from jax.experimental import pallas as pl
from jax.experimental.pallas import tpu as pltpu


def _infidelity_kernel(x_ref, attr_ref, out_ref, s_acc, death_acc, *, NP, PS, C, L):
    p = pl.program_id(1)

    a = attr_ref[0]
    pooled = jnp.concatenate(
        [jnp.max(a[:, q * PS:(q + 1) * PS], axis=1, keepdims=True) for q in range(NP)],
        axis=1,
    )
    pt = jnp.transpose(pooled)

    r = jnp.zeros((NP, PS), dtype=jnp.int32)
    sub = jax.lax.broadcasted_iota(jnp.int32, (NP, PS), 0)
    for q in range(NP):
        vq = pt[q:q + 1, :]
        cond = (vq < pt) | ((vq == pt) & (sub > q))
        r = r + cond.astype(jnp.int32)

    @pl.when(p == 0)
    def _():
        death_acc[...] = r

    @pl.when(p > 0)
    def _():
        death_acc[...] = jnp.minimum(death_acc[...], r)

    s_acc[pl.ds(p, 1), :] = jnp.sum(x_ref[0], axis=0, keepdims=True)

    @pl.when(p == NP - 1)
    def _():
        death = jnp.min(death_acc[...], axis=1, keepdims=True)
        stage = jax.lax.broadcasted_iota(jnp.int32, (NP, NP), 1)
        alive = (death >= stage).astype(jnp.float32)
        m = jax.lax.dot_general(
            alive, s_acc[...], (((0,), (0,)), ((), ())),
            preferred_element_type=jnp.float32,
        )
        m = m * (1.0 / C)

        m = m - jnp.max(m, axis=1, keepdims=True)
        e = jnp.exp(m)
        o = e / jnp.sum(e, axis=1, keepdims=True)

        u = 1.0 / L
        numer = jnp.sum(o[1:, :], axis=0, keepdims=True) + 0.5 * u
        res = (0.5 + numer / o[0:1, :]) * (1.0 / (NP + 1))
        out_ref[0] = res


@jax.jit
def kernel(x, attr, mask):
    B, C, L = x.shape
    PS = int(0.1 * L)
    NP = L // PS

    grid = (B, NP)
    out = pl.pallas_call(
        functools.partial(_infidelity_kernel, NP=NP, PS=PS, C=C, L=L),
        grid=grid,
        in_specs=[
            pl.BlockSpec((1, PS, L), lambda b, p: (b, p, 0)),
            pl.BlockSpec((1, PS, L), lambda b, p: (b, p, 0)),
        ],
        out_specs=pl.BlockSpec((1, 1, L), lambda b, p: (b, 0, 0)),
        out_shape=jax.ShapeDtypeStruct((B, 1, L), jnp.float32),
        scratch_shapes=[
            pltpu.VMEM((NP, L), jnp.float32),
            pltpu.VMEM((NP, PS), jnp.int32),
        ],
        compiler_params=pltpu.CompilerParams(
            dimension_semantics=("parallel", "arbitrary"),
        ),
    )(x, attr)
    return out.reshape(B, L)


# speedup vs baseline: 6117.4106x; 1.2723x over previous
import functools

import jax
import jax.numpy as jnp
import numpy as np
from jax.experimental import pallas as pl
from jax.experimental.pallas import tpu as pltpu


def _infidelity_kernel(x_ref, attr_ref, out_ref, s_acc, death_acc, *, NP, PS, KP, C, L):
    p = pl.program_id(1)
    RB = KP * PS

    a = attr_ref[0]
    pooled = jnp.concatenate(
        [jnp.max(a[:, q * PS:(q + 1) * PS], axis=1, keepdims=True) for q in range(NP)],
        axis=1,
    )
    pt = jnp.transpose(pooled)

    r = jnp.zeros((NP, RB), dtype=jnp.int32)
    sub = jax.lax.broadcasted_iota(jnp.int32, (NP, RB), 0)
    for q in range(NP):
        vq = pt[q:q + 1, :]
        cond = (vq < pt) | ((vq == pt) & (sub > q))
        r = r + cond.astype(jnp.int32)

    @pl.when(p == 0)
    def _():
        death_acc[...] = r

    @pl.when(p > 0)
    def _():
        death_acc[...] = jnp.minimum(death_acc[...], r)

    for j in range(KP):
        s_acc[pl.ds(p * KP + j, 1), :] = jnp.sum(
            x_ref[0, j * PS:(j + 1) * PS, :], axis=0, keepdims=True
        )

    @pl.when(p == (NP // KP) - 1)
    def _():
        death = jnp.min(death_acc[...], axis=1, keepdims=True)
        stage = jax.lax.broadcasted_iota(jnp.int32, (NP, NP), 1)
        alive = (death >= stage).astype(jnp.float32)
        m = jax.lax.dot_general(
            alive, s_acc[...], (((0,), (0,)), ((), ())),
            preferred_element_type=jnp.float32,
        )
        m = m * (1.0 / C)

        m = m - jnp.max(m, axis=1, keepdims=True)
        e = jnp.exp(m)
        o = e / jnp.sum(e, axis=1, keepdims=True)

        u = 1.0 / L
        numer = jnp.sum(o[1:, :], axis=0, keepdims=True) + 0.5 * u
        res = (0.5 + numer / o[0:1, :]) * (1.0 / (NP + 1))
        out_ref[0] = res


@jax.jit
def kernel(x, attr, mask):
    B, C, L = x.shape
    PS = int(0.1 * L)
    NP = L // PS
    KP = 2

    grid = (B, NP // KP)
    out = pl.pallas_call(
        functools.partial(_infidelity_kernel, NP=NP, PS=PS, KP=KP, C=C, L=L),
        grid=grid,
        in_specs=[
            pl.BlockSpec((1, KP * PS, L), lambda b, p: (b, p, 0)),
            pl.BlockSpec((1, KP * PS, L), lambda b, p: (b, p, 0)),
        ],
        out_specs=pl.BlockSpec((1, 1, L), lambda b, p: (b, 0, 0)),
        out_shape=jax.ShapeDtypeStruct((B, 1, L), jnp.float32),
        scratch_shapes=[
            pltpu.VMEM((NP, L), jnp.float32),
            pltpu.VMEM((NP, KP * PS), jnp.int32),
        ],
        compiler_params=pltpu.CompilerParams(
            dimension_semantics=("parallel", "arbitrary"),
        ),
    )(x, attr)
    return out.reshape(B, L)


# speedup vs baseline: 6541.1215x; 1.0693x over previous
import functools

import jax
import jax.numpy as jnp
import numpy as np
from jax.experimental import pallas as pl
from jax.experimental.pallas import tpu as pltpu


def _infidelity_kernel(x_ref, attr_ref, out_ref, s_acc, death_acc, *, NP, PS, KP, C, L):
    p = pl.program_id(1)
    RB = KP * PS

    a = attr_ref[0]
    pooled = jnp.concatenate(
        [jnp.max(a[:, q * PS:(q + 1) * PS], axis=1, keepdims=True) for q in range(NP)],
        axis=1,
    )
    pt = jnp.transpose(pooled)

    r = jnp.zeros((NP, RB), dtype=jnp.int32)
    sub = jax.lax.broadcasted_iota(jnp.int32, (NP, RB), 0)
    for q in range(NP):
        vq = pt[q:q + 1, :]
        cond = (vq < pt) | ((vq == pt) & (sub > q))
        r = r + cond.astype(jnp.int32)

    @pl.when(p == 0)
    def _():
        death_acc[...] = r

    @pl.when(p > 0)
    def _():
        death_acc[...] = jnp.minimum(death_acc[...], r)

    for j in range(KP):
        s_acc[pl.ds(p * KP + j, 1), :] = jnp.sum(
            x_ref[0, j * PS:(j + 1) * PS, :], axis=0, keepdims=True
        )

    @pl.when(p == (NP // KP) - 1)
    def _():
        death = jnp.min(death_acc[...], axis=1, keepdims=True)
        stage = jax.lax.broadcasted_iota(jnp.int32, (NP, NP), 1)
        alive = (death >= stage).astype(jnp.float32)
        m = jax.lax.dot_general(
            alive, s_acc[...], (((0,), (0,)), ((), ())),
            preferred_element_type=jnp.float32,
        )
        m = m * (1.0 / C)

        m = m - jnp.max(m, axis=1, keepdims=True)
        e = jnp.exp(m)
        o = e / jnp.sum(e, axis=1, keepdims=True)

        u = 1.0 / L
        numer = jnp.sum(o[1:, :], axis=0, keepdims=True) + 0.5 * u
        res = (0.5 + numer / o[0:1, :]) * (1.0 / (NP + 1))
        out_ref[0] = res


@jax.jit
def kernel(x, attr, mask):
    B, C, L = x.shape
    PS = int(0.1 * L)
    NP = L // PS
    KP = 5

    grid = (B, NP // KP)
    out = pl.pallas_call(
        functools.partial(_infidelity_kernel, NP=NP, PS=PS, KP=KP, C=C, L=L),
        grid=grid,
        in_specs=[
            pl.BlockSpec((1, KP * PS, L), lambda b, p: (b, p, 0)),
            pl.BlockSpec((1, KP * PS, L), lambda b, p: (b, p, 0)),
        ],
        out_specs=pl.BlockSpec((1, 1, L), lambda b, p: (b, 0, 0)),
        out_shape=jax.ShapeDtypeStruct((B, 1, L), jnp.float32),
        scratch_shapes=[
            pltpu.VMEM((NP, L), jnp.float32),
            pltpu.VMEM((NP, KP * PS), jnp.int32),
        ],
        compiler_params=pltpu.CompilerParams(
            dimension_semantics=("parallel", "arbitrary"),
        ),
    )(x, attr)
    return out.reshape(B, L)
